# baseline (speedup 1.0000x reference)
import functools

import jax
import jax.numpy as jnp
from jax import lax
from jax.experimental import pallas as pl
from jax.experimental.pallas import tpu as pltpu
from jax.experimental.pallas import tpu_sc as plsc

B = 16384
NUM_CORES = 2
NUM_SUBCORES = 16
LANES = 16
NUM_WORKERS = NUM_CORES * NUM_SUBCORES
CHUNK = B // NUM_WORKERS


@functools.partial(
    pl.kernel,
    mesh=plsc.VectorSubcoreMesh(core_axis_name="c", subcore_axis_name="s"),
    out_type=jax.ShapeDtypeStruct((B,), jnp.float32),
    scratch_types=[
        pltpu.VMEM((CHUNK,), jnp.float32),
        pltpu.VMEM((CHUNK,), jnp.int32),
        pltpu.VMEM((3 * LANES,), jnp.float32),
        pltpu.VMEM((CHUNK,), jnp.float32),
        pltpu.SemaphoreType.DMA,
    ],
)
def _sc_embed_scale(x_hbm, c_hbm, params_hbm, out_hbm, x_v, c_v, p_v, o_v, sem):
    wid = lax.axis_index("s") * NUM_CORES + lax.axis_index("c")
    base = wid * CHUNK
    cp_x = pltpu.async_copy(x_hbm.at[pl.ds(base, CHUNK)], x_v, sem)
    cp_c = pltpu.async_copy(c_hbm.at[pl.ds(base, CHUNK)], c_v, sem)
    cp_p = pltpu.async_copy(params_hbm, p_v, sem)
    cp_x.wait()
    cp_c.wait()
    cp_p.wait()
    t0 = p_v[pl.ds(0, LANES)]
    t1 = p_v[pl.ds(LANES, LANES)]
    sc = p_v[pl.ds(2 * LANES, LANES)]
    for i in range(CHUNK // LANES):
        sl = pl.ds(i * LANES, LANES)
        emb = jnp.where(c_v[sl] == 0, t0, t1)
        o_v[sl] = x_v[sl] * sc + emb
    pltpu.sync_copy(o_v, out_hbm.at[pl.ds(base, CHUNK)])


def kernel(x, x_cat, c, emb_table, scale):
    del x_cat
    c32 = c.reshape(-1).astype(jnp.int32)
    flat_table = emb_table.reshape(-1).astype(jnp.float32)
    params = jnp.concatenate(
        [
            jnp.broadcast_to(flat_table[0], (LANES,)),
            jnp.broadcast_to(flat_table[1], (LANES,)),
            jnp.broadcast_to(scale.astype(jnp.float32), (LANES,)),
        ]
    )
    return _sc_embed_scale(x.reshape(-1), c32, params)


# speedup vs baseline: 1.0702x; 1.0702x over previous
import functools

import jax
import jax.numpy as jnp
from jax import lax
from jax.experimental import pallas as pl
from jax.experimental.pallas import tpu as pltpu
from jax.experimental.pallas import tpu_sc as plsc

B = 16384
NUM_CORES = 2
NUM_SUBCORES = 16
LANES = 16
NUM_WORKERS = NUM_CORES * NUM_SUBCORES
CHUNK = B // NUM_WORKERS


@functools.partial(
    pl.kernel,
    mesh=plsc.VectorSubcoreMesh(core_axis_name="c", subcore_axis_name="s"),
    out_type=jax.ShapeDtypeStruct((B,), jnp.float32),
    scratch_types=[
        pltpu.VMEM((CHUNK,), jnp.float32),
        pltpu.VMEM((CHUNK,), jnp.int32),
        pltpu.VMEM((LANES,), jnp.float32),
        pltpu.VMEM((LANES,), jnp.float32),
        pltpu.VMEM((CHUNK,), jnp.float32),
        pltpu.SemaphoreType.DMA,
    ],
)
def _sc_embed_scale(x_hbm, c_hbm, t_hbm, s_hbm, out_hbm, x_v, c_v, t_v, s_v, o_v, sem):
    wid = lax.axis_index("s") * NUM_CORES + lax.axis_index("c")
    base = wid * CHUNK
    cp_x = pltpu.async_copy(x_hbm.at[pl.ds(base, CHUNK)], x_v, sem)
    cp_c = pltpu.async_copy(c_hbm.at[pl.ds(base, CHUNK)], c_v, sem)
    cp_t = pltpu.async_copy(t_hbm, t_v.at[pl.ds(0, 2)], sem)
    cp_s = pltpu.async_copy(s_hbm, s_v.at[pl.ds(0, 1)], sem)
    cp_x.wait()
    cp_c.wait()
    cp_t.wait()
    cp_s.wait()
    t_reg = t_v[...]
    zero_idx = jnp.zeros((LANES,), jnp.int32)
    sc_reg = s_v[...].at[zero_idx].get(mode="promise_in_bounds")
    for i in range(CHUNK // LANES):
        sl = pl.ds(i * LANES, LANES)
        emb = t_reg.at[c_v[sl]].get(mode="promise_in_bounds")
        o_v[sl] = x_v[sl] * sc_reg + emb
    pltpu.sync_copy(o_v, out_hbm.at[pl.ds(base, CHUNK)])


def kernel(x, x_cat, c, emb_table, scale):
    del x_cat
    c32 = c.reshape(-1).astype(jnp.int32)
    return _sc_embed_scale(
        x.reshape(-1),
        c32,
        emb_table.reshape(-1).astype(jnp.float32),
        jnp.reshape(scale, (1,)).astype(jnp.float32),
    )


# speedup vs baseline: 1.0931x; 1.0213x over previous
import functools

import jax
import jax.numpy as jnp
from jax import lax
from jax.experimental import pallas as pl
from jax.experimental.pallas import tpu as pltpu
from jax.experimental.pallas import tpu_sc as plsc

B = 16384
NUM_CORES = 2
NUM_SUBCORES = 16
LANES = 16
NUM_WORKERS = NUM_CORES * NUM_SUBCORES
CHUNK = B // NUM_WORKERS


@functools.partial(
    pl.kernel,
    mesh=plsc.VectorSubcoreMesh(core_axis_name="c", subcore_axis_name="s"),
    out_type=jax.ShapeDtypeStruct((B,), jnp.float32),
    scratch_types=[
        pltpu.VMEM((CHUNK,), jnp.float32),
        pltpu.VMEM((CHUNK,), jnp.int32),
        pltpu.VMEM((LANES,), jnp.float32),
        pltpu.VMEM((LANES,), jnp.float32),
        pltpu.VMEM((CHUNK,), jnp.float32),
        pltpu.SemaphoreType.DMA,
    ],
)
def _sc_embed_scale(x_hbm, c_hbm, t_hbm, s_hbm, out_hbm, x_v, c_v, t_v, s_v, o_v, sem):
    wid = lax.axis_index("s") * NUM_CORES + lax.axis_index("c")
    base = wid * CHUNK
    cp_x = pltpu.async_copy(x_hbm.at[pl.ds(base, CHUNK)], x_v, sem)
    cp_c = pltpu.async_copy(c_hbm.at[pl.ds(base, CHUNK)], c_v, sem)
    cp_t = pltpu.async_copy(t_hbm, t_v.at[pl.ds(0, 2)], sem)
    cp_s = pltpu.async_copy(s_hbm, s_v.at[pl.ds(0, 1)], sem)
    cp_x.wait()
    cp_c.wait()
    cp_t.wait()
    cp_s.wait()
    t_reg = t_v[...]
    zero_idx = jnp.zeros((LANES,), jnp.int32)
    sc_reg = s_v[...].at[zero_idx].get(mode="promise_in_bounds")

    def body(i, carry):
        sl = pl.ds(i * LANES, LANES)
        emb = t_reg.at[c_v[sl]].get(mode="promise_in_bounds")
        o_v[sl] = x_v[sl] * sc_reg + emb
        return carry

    lax.fori_loop(0, CHUNK // LANES, body, 0, unroll=1)
    pltpu.sync_copy(o_v, out_hbm.at[pl.ds(base, CHUNK)])


def kernel(x, x_cat, c, emb_table, scale):
    del x_cat
    c32 = c.reshape(-1).astype(jnp.int32)
    return _sc_embed_scale(
        x.reshape(-1),
        c32,
        emb_table.reshape(-1).astype(jnp.float32),
        jnp.reshape(scale, (1,)).astype(jnp.float32),
    )


# speedup vs baseline: 1.1546x; 1.0563x over previous
import functools

import jax
import jax.numpy as jnp
from jax import lax
from jax.experimental import pallas as pl
from jax.experimental.pallas import tpu as pltpu
from jax.experimental.pallas import tpu_sc as plsc

B = 16384
NUM_CORES = 1
NUM_SUBCORES = 16
LANES = 16
NUM_WORKERS = NUM_CORES * NUM_SUBCORES
CHUNK = B // NUM_WORKERS


@functools.partial(
    pl.kernel,
    mesh=plsc.VectorSubcoreMesh(core_axis_name="c", subcore_axis_name="s", num_cores=1),
    out_type=jax.ShapeDtypeStruct((B,), jnp.float32),
    scratch_types=[
        pltpu.VMEM((CHUNK,), jnp.float32),
        pltpu.VMEM((CHUNK,), jnp.int32),
        pltpu.VMEM((LANES,), jnp.float32),
        pltpu.VMEM((LANES,), jnp.float32),
        pltpu.VMEM((CHUNK,), jnp.float32),
        pltpu.SemaphoreType.DMA,
    ],
)
def _sc_embed_scale(x_hbm, c_hbm, t_hbm, s_hbm, out_hbm, x_v, c_v, t_v, s_v, o_v, sem):
    wid = lax.axis_index("s") * NUM_CORES + lax.axis_index("c")
    base = wid * CHUNK
    cp_x = pltpu.async_copy(x_hbm.at[pl.ds(base, CHUNK)], x_v, sem)
    cp_c = pltpu.async_copy(c_hbm.at[pl.ds(base, CHUNK)], c_v, sem)
    cp_t = pltpu.async_copy(t_hbm, t_v.at[pl.ds(0, 2)], sem)
    cp_s = pltpu.async_copy(s_hbm, s_v.at[pl.ds(0, 1)], sem)
    cp_x.wait()
    cp_c.wait()
    cp_t.wait()
    cp_s.wait()
    t_reg = t_v[...]
    zero_idx = jnp.zeros((LANES,), jnp.int32)
    sc_reg = s_v[...].at[zero_idx].get(mode="promise_in_bounds")

    def body(i, carry):
        sl = pl.ds(i * LANES, LANES)
        emb = t_reg.at[c_v[sl]].get(mode="promise_in_bounds")
        o_v[sl] = x_v[sl] * sc_reg + emb
        return carry

    lax.fori_loop(0, CHUNK // LANES, body, 0, unroll=1)
    pltpu.sync_copy(o_v, out_hbm.at[pl.ds(base, CHUNK)])


def kernel(x, x_cat, c, emb_table, scale):
    del x_cat
    c32 = c.reshape(-1).astype(jnp.int32)
    return _sc_embed_scale(
        x.reshape(-1),
        c32,
        emb_table.reshape(-1).astype(jnp.float32),
        jnp.reshape(scale, (1,)).astype(jnp.float32),
    )
